# Initial kernel scaffold; baseline (speedup 1.0000x reference)
import jax
import jax.numpy as jnp
from jax.experimental import pallas as pl


def kernel(x, codebook, W):
    raise NotImplementedError("write your pallas kernel here")


# speedup vs baseline: 1.2676x; 1.2676x over previous
import functools

import jax
import jax.numpy as jnp
from jax import lax
from jax.experimental import pallas as pl
from jax.experimental.pallas import tpu as pltpu
from jax.experimental.pallas import tpu_sc as plsc

DIM = 32
NCODES = 8192
NTOK = 8192
TBLK = 256
NT = NTOK // TBLK

NC, NS = 2, 16
NW = NC * NS
BPW = NTOK // NW
CHUNK = 128
NCH = BPW // CHUNK


def _ic_body(cb_ref, w_ref, ic_ref, icm2_ref):
    ic = lax.dot_general(cb_ref[...], w_ref[...],
                         (((1,), (1,)), ((), ())),
                         preferred_element_type=jnp.float32)
    ic_ref[...] = ic
    icm2_ref[...] = -2.0 * ic


def _tc_implicit_codebook(codebook, W):
    return pl.pallas_call(
        _ic_body,
        out_shape=[
            jax.ShapeDtypeStruct((NCODES, DIM), jnp.float32),
            jax.ShapeDtypeStruct((NCODES, DIM), jnp.float32),
        ],
    )(codebook, W)


def _nextf(v):
    b = lax.bitcast_convert_type(v, jnp.uint32)
    return lax.bitcast_convert_type(b + jnp.uint32(1), jnp.float32)


def _prevf(v):
    b = lax.bitcast_convert_type(v, jnp.uint32)
    return lax.bitcast_convert_type(b - jnp.uint32(1), jnp.float32)


def _argmin_body(x_ref, x2_ref, icm2_ref, c2_ref, idx_ref, loss_ref):
    i = pl.program_id(0)

    @pl.when(i == 0)
    def _init():
        loss_ref[0, 0] = 0.0

    x = x_ref[...]
    s2 = lax.dot_general(x, icm2_ref[...], (((1,), (1,)), ((), ())),
                         preferred_element_type=jnp.float32)
    d2 = (x2_ref[...] + c2_ref[...]) + s2
    dmin = jnp.min(d2, axis=1, keepdims=True)
    dminc = jnp.maximum(dmin, 0.0)
    r = jnp.sqrt(dminc)
    v = _nextf(r)
    v = v * v
    for _ in range(3):
        vn = _nextf(v)
        v = jnp.where(jnp.sqrt(vn) <= r, vn, v)
    for _ in range(5):
        v = jnp.where(jnp.sqrt(v) > r, _prevf(v), v)
    hi = v
    ii = lax.broadcasted_iota(jnp.int32, (TBLK, NCODES), 1)
    idx = jnp.min(jnp.where(d2 <= hi, ii, NCODES), axis=1, keepdims=True)
    idx_ref[...] = idx.reshape(1, TBLK, 1)
    loss_ref[0, 0] += jnp.sum(dminc) * (2.0 ** -18)


def _tc_argmin(x_flat, x2, icm2, c2):
    return pl.pallas_call(
        _argmin_body,
        grid=(NT,),
        in_specs=[
            pl.BlockSpec((TBLK, DIM), lambda i: (i, 0)),
            pl.BlockSpec((TBLK, 1), lambda i: (i, 0)),
            pl.BlockSpec((NCODES, DIM), lambda i: (0, 0)),
            pl.BlockSpec((1, NCODES), lambda i: (0, 0)),
        ],
        out_specs=[
            pl.BlockSpec((1, TBLK, 1), lambda i: (i, 0, 0)),
            pl.BlockSpec(memory_space=pltpu.SMEM),
        ],
        out_shape=[
            jax.ShapeDtypeStruct((NT, TBLK, 1), jnp.int32),
            jax.ShapeDtypeStruct((1, 1), jnp.float32),
        ],
    )(x_flat, x2, icm2, c2)


def _gather_body(idx_hbm, table_hbm, out_hbm, idx_v, rows_v, sem):
    wid = lax.axis_index("s") * NC + lax.axis_index("c")
    pltpu.sync_copy(idx_hbm.at[wid], idx_v)
    for c in range(NCH):
        pltpu.async_copy(table_hbm.at[idx_v.at[c]], rows_v.at[c], sem).wait()
    pltpu.sync_copy(rows_v, out_hbm.at[wid])


def _sc_gather(table, idx):
    mesh = plsc.VectorSubcoreMesh(core_axis_name="c", subcore_axis_name="s",
                                  num_cores=NC, num_subcores=NS)
    k = functools.partial(
        pl.kernel,
        mesh=mesh,
        compiler_params=pltpu.CompilerParams(use_tc_tiling_on_sc=False),
        out_type=jax.ShapeDtypeStruct((NW, NCH, CHUNK, DIM), jnp.float32),
        scratch_types=[
            pltpu.VMEM((NCH, CHUNK), jnp.int32),
            pltpu.VMEM((NCH, CHUNK, DIM), jnp.float32),
            pltpu.SemaphoreType.DMA,
        ],
    )(_gather_body)
    return k(idx.reshape(NW, NCH, CHUNK), table)


def kernel(x, codebook, W):
    x_flat = x.reshape(NTOK, DIM)
    ic, icm2 = _tc_implicit_codebook(codebook, W)
    x2 = jnp.sum(x ** 2, axis=-1, keepdims=True).reshape(NTOK, 1)
    c2 = jnp.sum(ic ** 2, axis=-1).reshape(1, NCODES)
    idx_out, loss_out = _tc_argmin(x_flat, x2, icm2, c2)
    idx_flat = idx_out.reshape(NTOK)
    quantized = _sc_gather(ic, idx_flat).reshape(x.shape)
    indices = idx_flat.reshape(x.shape[0], x.shape[1])
    commit_loss = loss_out[0, 0]
    return quantized, indices, commit_loss


# speedup vs baseline: 1.4078x; 1.1106x over previous
import functools

import jax
import jax.numpy as jnp
from jax import lax
from jax.experimental import pallas as pl
from jax.experimental.pallas import tpu as pltpu
from jax.experimental.pallas import tpu_sc as plsc

DIM = 32
NCODES = 8192
NTOK = 8192
TBLK = 512
NT = NTOK // TBLK

NC, NS = 2, 16
NW = NC * NS
BPW = NTOK // NW
CHUNK = 128
NCH = BPW // CHUNK


def _ic_body(cb_ref, w_ref, ic_ref, icm2_ref):
    ic = lax.dot_general(cb_ref[...], w_ref[...],
                         (((1,), (1,)), ((), ())),
                         preferred_element_type=jnp.float32)
    ic_ref[...] = ic
    icm2_ref[...] = -2.0 * ic


def _tc_implicit_codebook(codebook, W):
    return pl.pallas_call(
        _ic_body,
        out_shape=[
            jax.ShapeDtypeStruct((NCODES, DIM), jnp.float32),
            jax.ShapeDtypeStruct((NCODES, DIM), jnp.float32),
        ],
    )(codebook, W)


def _nextf(v):
    b = lax.bitcast_convert_type(v, jnp.uint32)
    return lax.bitcast_convert_type(b + jnp.uint32(1), jnp.float32)


def _prevf(v):
    b = lax.bitcast_convert_type(v, jnp.uint32)
    return lax.bitcast_convert_type(b - jnp.uint32(1), jnp.float32)


def _argmin_body(x_ref, x2_ref, icm2_ref, c2_ref, idx_ref, loss_ref):
    i = pl.program_id(0)

    @pl.when(i == 0)
    def _init():
        loss_ref[0, 0] = 0.0

    x = x_ref[...]
    s2 = lax.dot_general(x, icm2_ref[...], (((1,), (1,)), ((), ())),
                         preferred_element_type=jnp.float32)
    t = x2_ref[...] + c2_ref[...]
    dmin = jnp.min(t + s2, axis=1, keepdims=True)
    dminc = jnp.maximum(dmin, 0.0)
    r = jnp.sqrt(dminc)
    v = _nextf(r)
    v = v * v
    for _ in range(3):
        vn = _nextf(v)
        v = jnp.where(jnp.sqrt(vn) <= r, vn, v)
    for _ in range(5):
        v = jnp.where(jnp.sqrt(v) > r, _prevf(v), v)
    hi = v
    ii = lax.broadcasted_iota(jnp.int32, (TBLK, NCODES), 1)
    iif = lax.bitcast_convert_type(ii | jnp.int32(0x4B000000), jnp.float32)
    minf = jnp.min(jnp.where((s2 + t) <= hi, iif, jnp.float32(3e38)),
                   axis=1, keepdims=True)
    idx = lax.bitcast_convert_type(minf, jnp.int32) - jnp.int32(0x4B000000)
    idx_ref[...] = idx.reshape(1, TBLK, 1)
    loss_ref[0, 0] += jnp.sum(dminc) * (2.0 ** -18)


def _tc_argmin(x_flat, x2, icm2, c2):
    return pl.pallas_call(
        _argmin_body,
        grid=(NT,),
        in_specs=[
            pl.BlockSpec((TBLK, DIM), lambda i: (i, 0)),
            pl.BlockSpec((TBLK, 1), lambda i: (i, 0)),
            pl.BlockSpec((NCODES, DIM), lambda i: (0, 0)),
            pl.BlockSpec((1, NCODES), lambda i: (0, 0)),
        ],
        out_specs=[
            pl.BlockSpec((1, TBLK, 1), lambda i: (i, 0, 0)),
            pl.BlockSpec(memory_space=pltpu.SMEM),
        ],
        out_shape=[
            jax.ShapeDtypeStruct((NT, TBLK, 1), jnp.int32),
            jax.ShapeDtypeStruct((1, 1), jnp.float32),
        ],
    )(x_flat, x2, icm2, c2)


def _gather_body(idx_hbm, table_hbm, out_hbm, idx_v, rows_v, sem):
    wid = lax.axis_index("s") * NC + lax.axis_index("c")
    pltpu.sync_copy(idx_hbm.at[wid], idx_v)
    for c in range(NCH):
        pltpu.async_copy(table_hbm.at[idx_v.at[c]], rows_v.at[c], sem).wait()
    pltpu.sync_copy(rows_v, out_hbm.at[wid])


def _sc_gather(table, idx):
    mesh = plsc.VectorSubcoreMesh(core_axis_name="c", subcore_axis_name="s",
                                  num_cores=NC, num_subcores=NS)
    k = functools.partial(
        pl.kernel,
        mesh=mesh,
        compiler_params=pltpu.CompilerParams(use_tc_tiling_on_sc=False),
        out_type=jax.ShapeDtypeStruct((NW, NCH, CHUNK, DIM), jnp.float32),
        scratch_types=[
            pltpu.VMEM((NCH, CHUNK), jnp.int32),
            pltpu.VMEM((NCH, CHUNK, DIM), jnp.float32),
            pltpu.SemaphoreType.DMA,
        ],
    )(_gather_body)
    return k(idx.reshape(NW, NCH, CHUNK), table)


def kernel(x, codebook, W):
    x_flat = x.reshape(NTOK, DIM)
    ic, icm2 = _tc_implicit_codebook(codebook, W)
    x2 = jnp.sum(x ** 2, axis=-1, keepdims=True).reshape(NTOK, 1)
    c2 = jnp.sum(ic ** 2, axis=-1).reshape(1, NCODES)
    idx_out, loss_out = _tc_argmin(x_flat, x2, icm2, c2)
    idx_flat = idx_out.reshape(NTOK)
    quantized = _sc_gather(ic, idx_flat).reshape(x.shape)
    indices = idx_flat.reshape(x.shape[0], x.shape[1])
    commit_loss = loss_out[0, 0]
    return quantized, indices, commit_loss
